# baseline (speedup 1.0000x reference)
import functools

import jax
import jax.numpy as jnp
from jax import lax
from jax.experimental import pallas as pl
from jax.experimental.pallas import tpu as pltpu
from jax.experimental.pallas import tpu_sc as plsc

N_NODES = 10000
D = 256
DH = 128

NC = 2
NT = 16
K = 128
NCH = 80
EPT = NCH * K
E_PAD = NT * EPT

DEG_ROWS = 10112
DZR = DEG_ROWS // NT
TRASH = N_NODES

HALF = N_NODES // 2
ACC_ROWS = 5120
ZROWS = ACC_ROWS // NT
TRASH2 = HALF

BM = 1000
GRID = N_NODES // BM

_MESH = plsc.VectorSubcoreMesh(core_axis_name="c", subcore_axis_name="s")


@functools.partial(
    pl.kernel,
    mesh=_MESH,
    out_type=jax.ShapeDtypeStruct((DEG_ROWS, DH), jnp.float32),
    scratch_types=[
        pltpu.VMEM((NCH, K), jnp.int32),
        pltpu.VMEM((K, DH), jnp.float32),
        pltpu.VMEM_SHARED((DEG_ROWS, DH), jnp.float32),
    ],
)
def _deg_kernel(col_hbm, ones_hbm, zeros_hbm, out_hbm, colv, onesv, hist):
    c = lax.axis_index("c")
    s = lax.axis_index("s")

    @pl.when(c == 0)
    def _():
        pltpu.sync_copy(col_hbm.at[s], colv)
        pltpu.sync_copy(ones_hbm, onesv)
        pltpu.sync_copy(zeros_hbm, hist.at[pl.ds(s * DZR, DZR)])
        plsc.subcore_barrier()

        def body(j, carry):
            pltpu.sync_copy(onesv, hist.at[colv.at[j]], add=True)
            return carry

        lax.fori_loop(0, NCH, body, 0)
        plsc.subcore_barrier()
        pltpu.sync_copy(hist.at[pl.ds(s * DZR, DZR)],
                        out_hbm.at[pl.ds(s * DZR, DZR)])


@functools.partial(
    pl.kernel,
    mesh=_MESH,
    out_type=jax.ShapeDtypeStruct((NC, ACC_ROWS, DH), jnp.float32),
    scratch_types=[
        pltpu.VMEM((NCH, K), jnp.int32),
        pltpu.VMEM((NCH, K), jnp.int32),
        pltpu.VMEM((K, DH), jnp.float32),
        pltpu.VMEM((K, DH), jnp.float32),
        pltpu.SemaphoreType.DMA,
        pltpu.SemaphoreType.DMA,
        pltpu.VMEM_SHARED((ACC_ROWS, DH), jnp.float32),
    ],
)
def _agg_kernel(g_hbm, row2_hbm, col_hbm, zeros_hbm, out_hbm,
                rowv, colv, buf_a, buf_b, sem_a, sem_b, acc):
    c = lax.axis_index("c")
    s = lax.axis_index("s")
    pltpu.sync_copy(row2_hbm.at[c * NT + s], rowv)
    pltpu.sync_copy(col_hbm.at[s], colv)
    pltpu.sync_copy(zeros_hbm, acc.at[pl.ds(s * ZROWS, ZROWS)])
    plsc.subcore_barrier()

    pltpu.async_copy(g_hbm.at[rowv.at[0]], buf_a, sem_a)
    pltpu.async_copy(g_hbm.at[rowv.at[1]], buf_b, sem_b)

    def body(jj, carry):
        j0 = 2 * jj
        j1 = j0 + 1
        pltpu.make_async_copy(g_hbm.at[rowv.at[j0]], buf_a, sem_a).wait()
        pltpu.sync_copy(buf_a, acc.at[colv.at[j0]], add=True)

        @pl.when(jj < NCH // 2 - 1)
        def _():
            pltpu.async_copy(g_hbm.at[rowv.at[j0 + 2]], buf_a, sem_a)

        pltpu.make_async_copy(g_hbm.at[rowv.at[j1]], buf_b, sem_b).wait()
        pltpu.sync_copy(buf_b, acc.at[colv.at[j1]], add=True)

        @pl.when(jj < NCH // 2 - 1)
        def _():
            pltpu.async_copy(g_hbm.at[rowv.at[j1 + 2]], buf_b, sem_b)

        return carry

    lax.fori_loop(0, NCH // 2, body, 0)
    plsc.subcore_barrier()
    pltpu.sync_copy(acc.at[pl.ds(s * ZROWS, ZROWS)],
                    out_hbm.at[c].at[pl.ds(s * ZROWS, ZROWS)])


def _rows(i):
    return (i, 0)


def _mm_body(x_ref, w_ref, deg_ref, g_ref):
    dinv = lax.rsqrt(deg_ref[:, 0:1] + 1.0)
    g_ref[...] = jnp.dot(x_ref[...], w_ref[...],
                         preferred_element_type=jnp.float32) * dinv


def _mm_call(x, W, deg16):
    return pl.pallas_call(
        _mm_body,
        grid=(GRID,),
        in_specs=[
            pl.BlockSpec((BM, D), _rows),
            pl.BlockSpec((D, D), lambda i: (0, 0)),
            pl.BlockSpec((BM, 16), _rows),
        ],
        out_specs=pl.BlockSpec((BM, D), _rows),
        out_shape=jax.ShapeDtypeStruct((N_NODES, D), jnp.float32),
    )(x, W, deg16)


def _fused_body(s0_ref, s1_ref, g_ref, deg_ref, b_ref, w_ref, h_ref, g2_ref):
    dinv = lax.rsqrt(deg_ref[:, 0:1] + 1.0)
    S = jnp.concatenate([s0_ref[...], s1_ref[...]], axis=1)
    h = jnp.maximum(dinv * (S + g_ref[...]) + b_ref[...], 0.0)
    h_ref[...] = h
    g2_ref[...] = jnp.dot(h, w_ref[...],
                          preferred_element_type=jnp.float32) * dinv


def _fused_call(s0, s1, g, deg16, b, W2):
    return pl.pallas_call(
        _fused_body,
        grid=(GRID,),
        in_specs=[
            pl.BlockSpec((BM, DH), _rows),
            pl.BlockSpec((BM, DH), _rows),
            pl.BlockSpec((BM, D), _rows),
            pl.BlockSpec((BM, 16), _rows),
            pl.BlockSpec((1, D), lambda i: (0, 0)),
            pl.BlockSpec((D, D), lambda i: (0, 0)),
        ],
        out_specs=[
            pl.BlockSpec((BM, D), _rows),
            pl.BlockSpec((BM, D), _rows),
        ],
        out_shape=[
            jax.ShapeDtypeStruct((N_NODES, D), jnp.float32),
            jax.ShapeDtypeStruct((N_NODES, D), jnp.float32),
        ],
    )(s0, s1, g, deg16, b, W2)


def _epi_body(s0_ref, s1_ref, g_ref, deg_ref, b_ref, h_ref):
    dinv = lax.rsqrt(deg_ref[:, 0:1] + 1.0)
    S = jnp.concatenate([s0_ref[...], s1_ref[...]], axis=1)
    h_ref[...] = jnp.maximum(dinv * (S + g_ref[...]) + b_ref[...], 0.0)


def _epi_call(s0, s1, g, deg16, b):
    return pl.pallas_call(
        _epi_body,
        grid=(GRID,),
        in_specs=[
            pl.BlockSpec((BM, DH), _rows),
            pl.BlockSpec((BM, DH), _rows),
            pl.BlockSpec((BM, D), _rows),
            pl.BlockSpec((BM, 16), _rows),
            pl.BlockSpec((1, D), lambda i: (0, 0)),
        ],
        out_specs=pl.BlockSpec((BM, D), _rows),
        out_shape=jax.ShapeDtypeStruct((N_NODES, D), jnp.float32),
    )(s0, s1, g, deg16, b)


def _aggregate(g, row2, colh_a, colh_b, zeros_acc):
    g2d = g.reshape(2 * N_NODES, DH)
    Sa = _agg_kernel(g2d, row2, colh_a, zeros_acc)
    Sb = _agg_kernel(g2d, row2, colh_b, zeros_acc)
    S = jnp.concatenate([Sa[:, :HALF], Sb[:, :HALF]], axis=1)
    return S[0], S[1]


def kernel(x, edge_index, W1, b1, W2, b2):
    x = x.astype(jnp.float32)
    row = edge_index[0].astype(jnp.int32)
    col = edge_index[1].astype(jnp.int32)
    pad = E_PAD - row.shape[0]
    rowp = jnp.concatenate([row, jnp.zeros((pad,), jnp.int32)])
    colp = jnp.concatenate([col, jnp.full((pad,), TRASH, jnp.int32)])
    row2 = jnp.stack([rowp * 2, rowp * 2 + 1]).reshape(NC * NT, NCH, K)
    col3 = colp.reshape(NT, NCH, K)
    colh_a = jnp.where(colp < HALF, colp, TRASH2).reshape(NT, NCH, K)
    colh_b = jnp.where(colp >= HALF, colp - HALF, TRASH2).reshape(NT, NCH, K)

    ones_k = jnp.ones((K, DH), jnp.float32)
    zeros_deg = jnp.zeros((DZR, DH), jnp.float32)
    zeros_acc = jnp.zeros((ZROWS, DH), jnp.float32)

    deg16 = _deg_kernel(col3, ones_k, zeros_deg)[:N_NODES, :16]

    g1 = _mm_call(x, W1, deg16)
    s10, s11 = _aggregate(g1, row2, colh_a, colh_b, zeros_acc)
    h1, g2 = _fused_call(s10, s11, g1, deg16, b1.reshape(1, D), W2)
    s20, s21 = _aggregate(g2, row2, colh_a, colh_b, zeros_acc)
    h2 = _epi_call(s20, s21, g2, deg16, b2.reshape(1, D))
    return jnp.concatenate([h1, h2], axis=1)


# speedup vs baseline: 1.6274x; 1.6274x over previous
import functools

import jax
import jax.numpy as jnp
from jax import lax
from jax.experimental import pallas as pl
from jax.experimental.pallas import tpu as pltpu
from jax.experimental.pallas import tpu_sc as plsc

N_NODES = 10000
D = 256
DH = 128

NC = 2
NT = 16
K = 128
NCH = 80
EPT = NCH * K
E_PAD = NT * EPT

DEG_ROWS = 10112
DZR = DEG_ROWS // NT
TRASH = N_NODES

ACC_ROWS = 10112
ZROWS = ACC_ROWS // NT

BM = 1000
GRID = N_NODES // BM

_MESH = plsc.VectorSubcoreMesh(core_axis_name="c", subcore_axis_name="s")


@functools.partial(
    pl.kernel,
    mesh=_MESH,
    out_type=jax.ShapeDtypeStruct((DEG_ROWS, DH), jnp.float32),
    scratch_types=[
        pltpu.VMEM((NCH, K), jnp.int32),
        pltpu.VMEM((K, DH), jnp.float32),
        pltpu.VMEM_SHARED((DEG_ROWS, DH), jnp.float32),
    ],
)
def _deg_kernel(col_hbm, ones_hbm, zeros_hbm, out_hbm, colv, onesv, hist):
    c = lax.axis_index("c")
    s = lax.axis_index("s")

    @pl.when(c == 0)
    def _():
        pltpu.sync_copy(col_hbm.at[s], colv)
        pltpu.sync_copy(ones_hbm, onesv)
        pltpu.sync_copy(zeros_hbm, hist.at[pl.ds(s * DZR, DZR)])
        plsc.subcore_barrier()

        def body(j, carry):
            pltpu.sync_copy(onesv, hist.at[colv.at[j]], add=True)
            return carry

        lax.fori_loop(0, NCH, body, 0)
        plsc.subcore_barrier()
        pltpu.sync_copy(hist.at[pl.ds(s * DZR, DZR)],
                        out_hbm.at[pl.ds(s * DZR, DZR)])


@functools.partial(
    pl.kernel,
    mesh=_MESH,
    out_type=jax.ShapeDtypeStruct((NC, ACC_ROWS, DH), jnp.float32),
    scratch_types=[
        pltpu.VMEM((NCH, K), jnp.int32),
        pltpu.VMEM((NCH, K), jnp.int32),
        pltpu.VMEM((K, DH), jnp.float32),
        pltpu.SemaphoreType.DMA,
        pltpu.VMEM_SHARED((ACC_ROWS, DH), jnp.float32),
    ],
)
def _agg_kernel(g_hbm, row2_hbm, col_hbm, zeros_hbm, out_hbm,
                rowv, colv, buf, sem, acc):
    c = lax.axis_index("c")
    s = lax.axis_index("s")
    pltpu.sync_copy(row2_hbm.at[c * NT + s], rowv)
    pltpu.sync_copy(col_hbm.at[s], colv)
    pltpu.sync_copy(zeros_hbm, acc.at[pl.ds(s * ZROWS, ZROWS)])
    plsc.subcore_barrier()

    def body(j, carry):
        pltpu.async_copy(g_hbm.at[rowv.at[j]], buf, sem).wait()
        pltpu.sync_copy(buf, acc.at[colv.at[j]], add=True)
        return carry

    lax.fori_loop(0, NCH, body, 0)
    plsc.subcore_barrier()
    pltpu.sync_copy(acc.at[pl.ds(s * ZROWS, ZROWS)],
                    out_hbm.at[c].at[pl.ds(s * ZROWS, ZROWS)])


def _rows(i):
    return (i, 0)


def _mm_body(x_ref, w_ref, h_ref):
    h_ref[...] = jnp.dot(x_ref[...], w_ref[...],
                         preferred_element_type=jnp.float32)


def _mm_call(x, W):
    return pl.pallas_call(
        _mm_body,
        grid=(GRID,),
        in_specs=[
            pl.BlockSpec((BM, D), _rows),
            pl.BlockSpec((D, D), lambda i: (0, 0)),
        ],
        out_specs=pl.BlockSpec((BM, D), _rows),
        out_shape=jax.ShapeDtypeStruct((N_NODES, D), jnp.float32),
    )(x, W)


def _scale_body(h_ref, deg_ref, g_ref):
    dinv = lax.rsqrt(deg_ref[:, 0:1] + 1.0)
    g_ref[...] = h_ref[...] * dinv


def _scale_call(h, deg16):
    return pl.pallas_call(
        _scale_body,
        grid=(GRID,),
        in_specs=[
            pl.BlockSpec((BM, D), _rows),
            pl.BlockSpec((BM, 16), _rows),
        ],
        out_specs=pl.BlockSpec((BM, D), _rows),
        out_shape=jax.ShapeDtypeStruct((N_NODES, D), jnp.float32),
    )(h, deg16)


def _fused_body(s0_ref, s1_ref, g_ref, deg_ref, b_ref, w_ref, h_ref, g2_ref):
    dinv = lax.rsqrt(deg_ref[:, 0:1] + 1.0)
    S = jnp.concatenate([s0_ref[...], s1_ref[...]], axis=1)
    h = jnp.maximum(dinv * (S + g_ref[...]) + b_ref[...], 0.0)
    h_ref[...] = h
    g2_ref[...] = jnp.dot(h, w_ref[...],
                          preferred_element_type=jnp.float32) * dinv


def _fused_call(s0, s1, g, deg16, b, W2):
    return pl.pallas_call(
        _fused_body,
        grid=(GRID,),
        in_specs=[
            pl.BlockSpec((BM, DH), _rows),
            pl.BlockSpec((BM, DH), _rows),
            pl.BlockSpec((BM, D), _rows),
            pl.BlockSpec((BM, 16), _rows),
            pl.BlockSpec((1, D), lambda i: (0, 0)),
            pl.BlockSpec((D, D), lambda i: (0, 0)),
        ],
        out_specs=[
            pl.BlockSpec((BM, D), _rows),
            pl.BlockSpec((BM, D), _rows),
        ],
        out_shape=[
            jax.ShapeDtypeStruct((N_NODES, D), jnp.float32),
            jax.ShapeDtypeStruct((N_NODES, D), jnp.float32),
        ],
    )(s0, s1, g, deg16, b, W2)


def _epi_body(s0_ref, s1_ref, g_ref, deg_ref, b_ref, h_ref):
    dinv = lax.rsqrt(deg_ref[:, 0:1] + 1.0)
    S = jnp.concatenate([s0_ref[...], s1_ref[...]], axis=1)
    h_ref[...] = jnp.maximum(dinv * (S + g_ref[...]) + b_ref[...], 0.0)


def _epi_call(s0, s1, g, deg16, b):
    return pl.pallas_call(
        _epi_body,
        grid=(GRID,),
        in_specs=[
            pl.BlockSpec((BM, DH), _rows),
            pl.BlockSpec((BM, DH), _rows),
            pl.BlockSpec((BM, D), _rows),
            pl.BlockSpec((BM, 16), _rows),
            pl.BlockSpec((1, D), lambda i: (0, 0)),
        ],
        out_specs=pl.BlockSpec((BM, D), _rows),
        out_shape=jax.ShapeDtypeStruct((N_NODES, D), jnp.float32),
    )(s0, s1, g, deg16, b)


def kernel(x, edge_index, W1, b1, W2, b2):
    x = x.astype(jnp.float32)
    row = edge_index[0].astype(jnp.int32)
    col = edge_index[1].astype(jnp.int32)
    pad = E_PAD - row.shape[0]
    rowp = jnp.concatenate([row, jnp.zeros((pad,), jnp.int32)])
    colp = jnp.concatenate([col, jnp.full((pad,), TRASH, jnp.int32)])
    row2 = jnp.stack([rowp * 2, rowp * 2 + 1]).reshape(NC * NT, NCH, K)
    col3 = colp.reshape(NT, NCH, K)

    ones_k = jnp.ones((K, DH), jnp.float32)
    zeros_deg = jnp.zeros((DZR, DH), jnp.float32)
    zeros_acc = jnp.zeros((ZROWS, DH), jnp.float32)

    deg16 = _deg_kernel(col3, ones_k, zeros_deg)[:N_NODES, :16]
    h1r = _mm_call(x, W1)
    g1 = _scale_call(h1r, deg16)

    S1 = _agg_kernel(g1.reshape(2 * N_NODES, DH), row2, col3, zeros_acc)
    h1, g2 = _fused_call(S1[0, :N_NODES], S1[1, :N_NODES], g1, deg16,
                         b1.reshape(1, D), W2)
    S2 = _agg_kernel(g2.reshape(2 * N_NODES, DH), row2, col3, zeros_acc)
    h2 = _epi_call(S2[0, :N_NODES], S2[1, :N_NODES], g2, deg16,
                   b2.reshape(1, D))
    return jnp.concatenate([h1, h2], axis=1)


# speedup vs baseline: 1.7373x; 1.0675x over previous
import functools

import jax
import jax.numpy as jnp
from jax import lax
from jax.experimental import pallas as pl
from jax.experimental.pallas import tpu as pltpu
from jax.experimental.pallas import tpu_sc as plsc

N_NODES = 10000
D = 256
DH = 128

NC = 2
NT = 16
K = 128
NCH = 80
EPT = NCH * K
E_PAD = NT * EPT

DEG_ROWS = 10112
DZR = DEG_ROWS // NT
TRASH = N_NODES

ACC_ROWS = 10112
ZROWS = ACC_ROWS // NT

BM = 1000
GRID = N_NODES // BM

_MESH = plsc.VectorSubcoreMesh(core_axis_name="c", subcore_axis_name="s")


@functools.partial(
    pl.kernel,
    mesh=_MESH,
    out_type=jax.ShapeDtypeStruct((DEG_ROWS, DH), jnp.float32),
    scratch_types=[
        pltpu.VMEM((NCH, K), jnp.int32),
        pltpu.VMEM((K, DH), jnp.float32),
        pltpu.VMEM_SHARED((DEG_ROWS, DH), jnp.float32),
    ],
)
def _deg_kernel(col_hbm, ones_hbm, zeros_hbm, out_hbm, colv, onesv, hist):
    c = lax.axis_index("c")
    s = lax.axis_index("s")

    @pl.when(c == 0)
    def _():
        pltpu.sync_copy(col_hbm.at[s], colv)
        pltpu.sync_copy(ones_hbm, onesv)
        pltpu.sync_copy(zeros_hbm, hist.at[pl.ds(s * DZR, DZR)])
        plsc.subcore_barrier()

        def body(j, carry):
            pltpu.sync_copy(onesv, hist.at[colv.at[j]], add=True)
            return carry

        lax.fori_loop(0, NCH, body, 0)
        plsc.subcore_barrier()
        pltpu.sync_copy(hist.at[pl.ds(s * DZR, DZR)],
                        out_hbm.at[pl.ds(s * DZR, DZR)])


@functools.partial(
    pl.kernel,
    mesh=_MESH,
    out_type=jax.ShapeDtypeStruct((NC, ACC_ROWS, DH), jnp.float32),
    scratch_types=[
        pltpu.VMEM((NCH // 2, K), jnp.int32),
        pltpu.VMEM((NCH // 2, K), jnp.int32),
        pltpu.VMEM((K, DH), jnp.float32),
        pltpu.VMEM((K, DH), jnp.float32),
        pltpu.SemaphoreType.DMA,
        pltpu.SemaphoreType.DMA,
        pltpu.VMEM_SHARED((ACC_ROWS, DH), jnp.float32),
    ],
)
def _agg_kernel(g_hbm, row2_hbm, col_hbm, init_hbm, out_hbm,
                rowv, colv, buf_a, buf_b, sem_a, sem_b, acc):
    NH = NCH // 2
    c = lax.axis_index("c")
    s = lax.axis_index("s")
    pltpu.sync_copy(row2_hbm.at[c * NT + s], rowv)
    pltpu.sync_copy(col_hbm.at[s], colv)
    pltpu.sync_copy(init_hbm.at[c].at[pl.ds(s * ZROWS, ZROWS)],
                    acc.at[pl.ds(s * ZROWS, ZROWS)])
    plsc.subcore_barrier()

    pltpu.async_copy(g_hbm.at[rowv.at[0]], buf_a, sem_a)
    pltpu.async_copy(g_hbm.at[rowv.at[1]], buf_b, sem_b)

    def body(jj, carry):
        j0 = 2 * jj
        j1 = j0 + 1
        pltpu.make_async_copy(g_hbm.at[rowv.at[j0]], buf_a, sem_a).wait()
        pltpu.sync_copy(buf_a, acc.at[colv.at[j0]], add=True)

        @pl.when(jj < NH // 2 - 1)
        def _():
            pltpu.async_copy(g_hbm.at[rowv.at[j0 + 2]], buf_a, sem_a)

        pltpu.make_async_copy(g_hbm.at[rowv.at[j1]], buf_b, sem_b).wait()
        pltpu.sync_copy(buf_b, acc.at[colv.at[j1]], add=True)

        @pl.when(jj < NH // 2 - 1)
        def _():
            pltpu.async_copy(g_hbm.at[rowv.at[j1 + 2]], buf_b, sem_b)

        return carry

    lax.fori_loop(0, NH // 2, body, 0)
    plsc.subcore_barrier()
    pltpu.sync_copy(acc.at[pl.ds(s * ZROWS, ZROWS)],
                    out_hbm.at[c].at[pl.ds(s * ZROWS, ZROWS)])


def _rows(i):
    return (i, 0)


def _mm_body(x_ref, w_ref, h_ref):
    h_ref[...] = jnp.dot(x_ref[...], w_ref[...],
                         preferred_element_type=jnp.float32)


def _mm_call(x, W):
    return pl.pallas_call(
        _mm_body,
        grid=(GRID,),
        in_specs=[
            pl.BlockSpec((BM, D), _rows),
            pl.BlockSpec((D, D), lambda i: (0, 0)),
        ],
        out_specs=pl.BlockSpec((BM, D), _rows),
        out_shape=jax.ShapeDtypeStruct((N_NODES, D), jnp.float32),
    )(x, W)


def _scale_body(h_ref, deg_ref, g_ref):
    dinv = lax.rsqrt(deg_ref[:, 0:1] + 1.0)
    g_ref[...] = h_ref[...] * dinv


def _scale_call(h, deg16):
    return pl.pallas_call(
        _scale_body,
        grid=(GRID,),
        in_specs=[
            pl.BlockSpec((BM, D), _rows),
            pl.BlockSpec((BM, 16), _rows),
        ],
        out_specs=pl.BlockSpec((BM, D), _rows),
        out_shape=jax.ShapeDtypeStruct((N_NODES, D), jnp.float32),
    )(h, deg16)


def _fused_body(s0_ref, s1_ref, g_ref, deg_ref, b_ref, w_ref, h_ref, g2_ref):
    dinv = lax.rsqrt(deg_ref[:, 0:1] + 1.0)
    S = jnp.concatenate([s0_ref[...], s1_ref[...]], axis=1)
    h = jnp.maximum(dinv * (S + g_ref[...]) + b_ref[...], 0.0)
    h_ref[...] = h
    g2_ref[...] = jnp.dot(h, w_ref[...],
                          preferred_element_type=jnp.float32) * dinv


def _fused_call(s0, s1, g, deg16, b, W2):
    return pl.pallas_call(
        _fused_body,
        grid=(GRID,),
        in_specs=[
            pl.BlockSpec((BM, DH), _rows),
            pl.BlockSpec((BM, DH), _rows),
            pl.BlockSpec((BM, D), _rows),
            pl.BlockSpec((BM, 16), _rows),
            pl.BlockSpec((1, D), lambda i: (0, 0)),
            pl.BlockSpec((D, D), lambda i: (0, 0)),
        ],
        out_specs=[
            pl.BlockSpec((BM, D), _rows),
            pl.BlockSpec((BM, D), _rows),
        ],
        out_shape=[
            jax.ShapeDtypeStruct((N_NODES, D), jnp.float32),
            jax.ShapeDtypeStruct((N_NODES, D), jnp.float32),
        ],
    )(s0, s1, g, deg16, b, W2)


def _epi_body(s0_ref, s1_ref, g_ref, deg_ref, b_ref, h_ref):
    dinv = lax.rsqrt(deg_ref[:, 0:1] + 1.0)
    S = jnp.concatenate([s0_ref[...], s1_ref[...]], axis=1)
    h_ref[...] = jnp.maximum(dinv * (S + g_ref[...]) + b_ref[...], 0.0)


def _epi_call(s0, s1, g, deg16, b):
    return pl.pallas_call(
        _epi_body,
        grid=(GRID,),
        in_specs=[
            pl.BlockSpec((BM, DH), _rows),
            pl.BlockSpec((BM, DH), _rows),
            pl.BlockSpec((BM, D), _rows),
            pl.BlockSpec((BM, 16), _rows),
            pl.BlockSpec((1, D), lambda i: (0, 0)),
        ],
        out_specs=pl.BlockSpec((BM, D), _rows),
        out_shape=jax.ShapeDtypeStruct((N_NODES, D), jnp.float32),
    )(s0, s1, g, deg16, b)


def kernel(x, edge_index, W1, b1, W2, b2):
    x = x.astype(jnp.float32)
    row = edge_index[0].astype(jnp.int32)
    col = edge_index[1].astype(jnp.int32)
    pad = E_PAD - row.shape[0]
    rowp = jnp.concatenate([row, jnp.zeros((pad,), jnp.int32)])
    colp = jnp.concatenate([col, jnp.full((pad,), TRASH, jnp.int32)])
    row2 = jnp.stack([rowp * 2, rowp * 2 + 1]).reshape(NC * NT, NCH, K)
    col3 = colp.reshape(NT, NCH, K)

    NH = NCH // 2
    row2a, row2b = row2[:, :NH], row2[:, NH:]
    col3a, col3b = col3[:, :NH], col3[:, NH:]

    ones_k = jnp.ones((K, DH), jnp.float32)
    zeros_deg = jnp.zeros((DZR, DH), jnp.float32)
    zeros_init = jnp.zeros((NC, ACC_ROWS, DH), jnp.float32)

    deg16 = _deg_kernel(col3, ones_k, zeros_deg)[:N_NODES, :16]
    h1r = _mm_call(x, W1)
    g1 = _scale_call(h1r, deg16)

    g1r = g1.reshape(2 * N_NODES, DH)
    S1 = _agg_kernel(g1r, row2b, col3b,
                     _agg_kernel(g1r, row2a, col3a, zeros_init))
    h1, g2 = _fused_call(S1[0, :N_NODES], S1[1, :N_NODES], g1, deg16,
                         b1.reshape(1, D), W2)
    g2r = g2.reshape(2 * N_NODES, DH)
    S2 = _agg_kernel(g2r, row2b, col3b,
                     _agg_kernel(g2r, row2a, col3a, zeros_init))
    h2 = _epi_call(S2[0, :N_NODES], S2[1, :N_NODES], g2, deg16,
                   b2.reshape(1, D))
    return jnp.concatenate([h1, h2], axis=1)
